# Initial kernel scaffold; baseline (speedup 1.0000x reference)
import jax
import jax.numpy as jnp
from jax.experimental import pallas as pl


def kernel(user_table, item_table, user_list, pos_items, neg_items):
    raise NotImplementedError("write your pallas kernel here")


# speedup vs baseline: 1.0486x; 1.0486x over previous
import functools

import jax
import jax.numpy as jnp
from jax import lax
from jax.experimental import pallas as pl
from jax.experimental.pallas import tpu as pltpu
from jax.experimental.pallas import tpu_sc as plsc

N_USERS = 100000
N_ITEMS = 100000
DIM = 128
BATCH = 16384

LANES = 16
NUM_WORKERS = 32
B_PER_W = BATCH // NUM_WORKERS
CHUNK = 128
N_CHUNKS = B_PER_W // CHUNK


def _mf_kernel(user_table, item_table, user_list, pos_items, neg_items,
               user_out, pos_out, neg_out, partials,
               idx_v, rows_v, acc_v, sem):
    nc = 2
    wid = lax.axis_index("s") * nc + lax.axis_index("c")
    base = wid * B_PER_W

    acc = jnp.zeros((LANES,), jnp.float32)

    for table, idx_hbm, out_hbm in (
        (user_table, user_list, user_out),
        (item_table, pos_items, pos_out),
        (item_table, neg_items, neg_out),
    ):
        for c in range(N_CHUNKS):
            off = base + c * CHUNK
            pltpu.sync_copy(idx_hbm.at[pl.ds(off, CHUNK)], idx_v)
            pltpu.async_copy(table.at[idx_v], rows_v, sem).wait()
            pltpu.sync_copy(rows_v, out_hbm.at[pl.ds(off, CHUNK)])

            def body(r, a):
                for cc in range(DIM // LANES):
                    v = rows_v[r, pl.ds(cc * LANES, LANES)]
                    a = a + v * v
                return a

            acc = lax.fori_loop(0, CHUNK, body, acc)

    acc_v[...] = acc
    pltpu.sync_copy(acc_v, partials.at[wid])


@jax.jit
def kernel(user_table, item_table, user_list, pos_items, neg_items):
    mesh = plsc.VectorSubcoreMesh(core_axis_name="c", subcore_axis_name="s")
    f = functools.partial(
        pl.kernel,
        mesh=mesh,
        out_type=[
            jax.ShapeDtypeStruct((BATCH, DIM), jnp.float32),
            jax.ShapeDtypeStruct((BATCH, DIM), jnp.float32),
            jax.ShapeDtypeStruct((BATCH, DIM), jnp.float32),
            jax.ShapeDtypeStruct((NUM_WORKERS, LANES), jnp.float32),
        ],
        scratch_types=[
            pltpu.VMEM((CHUNK,), jnp.int32),
            pltpu.VMEM((CHUNK, DIM), jnp.float32),
            pltpu.VMEM((LANES,), jnp.float32),
            pltpu.SemaphoreType.DMA,
        ],
    )(_mf_kernel)
    user_emb, posI_emb, negI_emb, partials = f(
        user_table, item_table,
        user_list.astype(jnp.int32),
        pos_items.astype(jnp.int32),
        neg_items.astype(jnp.int32),
    )
    reg = jnp.sum(partials) / jnp.float32(BATCH)
    return (user_emb, posI_emb, negI_emb, reg)


# speedup vs baseline: 1.7149x; 1.6355x over previous
import functools

import jax
import jax.numpy as jnp
from jax import lax
from jax.experimental import pallas as pl
from jax.experimental.pallas import tpu as pltpu
from jax.experimental.pallas import tpu_sc as plsc

N_USERS = 100000
N_ITEMS = 100000
DIM = 128
BATCH = 16384

LANES = 16
NUM_WORKERS = 32
B_PER_W = BATCH // NUM_WORKERS
CHUNK = 128
N_CHUNKS = B_PER_W // CHUNK


NBUF = 4


def _mf_kernel(user_table, item_table, user_list, pos_items, neg_items,
               user_out, pos_out, neg_out, partials,
               idx_all, bufs, acc_v,
               isem, g0, g1, g2, g3, w0, w1, w2, w3):
    nc = 2
    wid = lax.axis_index("s") * nc + lax.axis_index("c")
    base = wid * B_PER_W
    gsem = (g0, g1, g2, g3)
    wsem = (w0, w1, w2, w3)

    chunks = []
    for table, idx_hbm, out_hbm in (
        (user_table, user_list, user_out),
        (item_table, pos_items, pos_out),
        (item_table, neg_items, neg_out),
    ):
        for c in range(N_CHUNKS):
            chunks.append((table, idx_hbm, out_hbm, base + c * CHUNK))

    idescs = [
        pltpu.async_copy(idx_hbm.at[pl.ds(off, CHUNK)], idx_all.at[j], isem)
        for j, (_, idx_hbm, _, off) in enumerate(chunks)
    ]
    for d in idescs:
        d.wait()

    def gather(g, b):
        table = chunks[g][0]
        return pltpu.async_copy(table.at[idx_all.at[g]], bufs.at[b], gsem[b])

    gdescs = [None] * len(chunks)
    for g in range(NBUF):
        gdescs[g] = gather(g, g)

    a0 = jnp.zeros((LANES,), jnp.float32)
    a1 = jnp.zeros((LANES,), jnp.float32)

    for g in range(len(chunks)):
        b = g % NBUF
        out_hbm, off = chunks[g][2], chunks[g][3]
        gdescs[g].wait()
        wdesc = pltpu.async_copy(bufs.at[b], out_hbm.at[pl.ds(off, CHUNK)],
                                 wsem[b])

        def body(r, accs, b=b):
            x0, x1 = accs
            for cc in range(4):
                v = bufs[b, r, pl.ds(cc * LANES, LANES)]
                x0 = x0 + v * v
            for cc in range(4, 8):
                v = bufs[b, r, pl.ds(cc * LANES, LANES)]
                x1 = x1 + v * v
            return (x0, x1)

        a0, a1 = lax.fori_loop(0, CHUNK, body, (a0, a1))
        wdesc.wait()
        if g + NBUF < len(chunks):
            gdescs[g + NBUF] = gather(g + NBUF, b)

    acc_v[...] = a0 + a1
    pltpu.sync_copy(acc_v, partials.at[wid])


@jax.jit
def kernel(user_table, item_table, user_list, pos_items, neg_items):
    mesh = plsc.VectorSubcoreMesh(core_axis_name="c", subcore_axis_name="s")
    f = functools.partial(
        pl.kernel,
        mesh=mesh,
        out_type=[
            jax.ShapeDtypeStruct((BATCH, DIM), jnp.float32),
            jax.ShapeDtypeStruct((BATCH, DIM), jnp.float32),
            jax.ShapeDtypeStruct((BATCH, DIM), jnp.float32),
            jax.ShapeDtypeStruct((NUM_WORKERS, LANES), jnp.float32),
        ],
        scratch_types=[
            pltpu.VMEM((3 * N_CHUNKS, CHUNK), jnp.int32),
            pltpu.VMEM((NBUF, CHUNK, DIM), jnp.float32),
            pltpu.VMEM((LANES,), jnp.float32),
        ] + [pltpu.SemaphoreType.DMA] * 9,
    )(_mf_kernel)
    user_emb, posI_emb, negI_emb, partials = f(
        user_table, item_table,
        user_list.astype(jnp.int32),
        pos_items.astype(jnp.int32),
        neg_items.astype(jnp.int32),
    )
    reg = jnp.sum(partials) / jnp.float32(BATCH)
    return (user_emb, posI_emb, negI_emb, reg)
